# baseline (speedup 1.0000x reference)
import functools

import jax
import jax.numpy as jnp
from jax import lax
from jax.experimental import pallas as pl
from jax.experimental.pallas import tpu as pltpu
from jax.experimental.pallas import tpu_sc as plsc

_NC = 2
_NS = 16
_NW = _NC * _NS
_LANES = 16
_CHUNK = 48


def _sc_segment_partials(sent_embs, cu_bcast, row_begin):
    total, d = sent_embs.shape
    nseg = cu_bcast.shape[0]
    c = _CHUNK
    rpw = -(-(total - row_begin) // _NW)
    mesh = plsc.VectorSubcoreMesh(core_axis_name="c", subcore_axis_name="s")

    @functools.partial(
        pl.kernel,
        out_type=jax.ShapeDtypeStruct((_NW, nseg * d), jnp.float32),
        mesh=mesh,
        compiler_params=pltpu.CompilerParams(needs_layout_passes=False),
        scratch_types=[
            pltpu.VMEM(((nseg + 1) * d,), jnp.float32),
            pltpu.VMEM((nseg, _LANES), jnp.int32),
            pltpu.VMEM((c, d), jnp.float32),
            pltpu.VMEM((c, d), jnp.float32),
            pltpu.VMEM((c,), jnp.int32),
            pltpu.VMEM((c,), jnp.int32),
            pltpu.SemaphoreType.DMA,
            pltpu.SemaphoreType.DMA,
        ],
    )
    def sc_kernel(x_hbm, cu_hbm, out_hbm, acc, cu_b, buf0, buf1,
                  idx0, idx1, sem0, sem1):
        wid = lax.axis_index("s") * _NC + lax.axis_index("c")
        r0 = row_begin + wid * rpw
        r1 = jnp.minimum(total, r0 + rpw)
        nrows = jnp.maximum(r1 - r0, 0)
        nch = (nrows + c - 1) // c

        pltpu.sync_copy(cu_hbm, cu_b)
        lane_iota = lax.iota(jnp.int32, _LANES)
        c_v = jnp.full((_LANES,), c, jnp.int32)
        zero_f = jnp.zeros((_LANES,), jnp.float32)

        def zero_body(i, _):
            for u in range(16):
                acc[pl.ds((i * 16 + u) * _LANES, _LANES)] = zero_f
            return 0
        lax.fori_loop(0, (nseg + 1) * d // (_LANES * 16), zero_body, 0)

        def start(idx_ref, buf, sem, k):
            s0 = r0 + k * c
            for j in range(c // _LANES):
                idx_ref[pl.ds(j * _LANES, _LANES)] = jnp.minimum(
                    s0 + j * _LANES + lane_iota, total - 1)
            pltpu.async_copy(x_hbm.at[idx_ref], buf, sem)

        def drain(buf, sem):
            pltpu.make_async_copy(x_hbm.at[pl.ds(0, c)], buf, sem).wait()

        def process(buf, k):
            s0 = r0 + k * c
            s0_v = jnp.full((_LANES,), s0, jnp.int32)
            seg1_v = jnp.zeros((_LANES,), jnp.int32)
            t_v = c_v
            for j in range(nseg):
                cuj = cu_b[j, :]
                gt = cuj > s0_v
                seg1_v = seg1_v + jnp.where(gt, 0, 1)
                t_v = jnp.minimum(t_v, jnp.where(gt, cuj - s0_v, c_v))
            base1_v = seg1_v * d + lane_iota
            base2_v = jnp.minimum(seg1_v + 1, nseg) * d + lane_iota

            def interior():
                strag_row = jnp.maximum(t_v - 1, 0)
                odd = (t_v & 1) == 1

                def cc_body(cc, _):
                    o = cc * _LANES
                    s_head = zero_f
                    s_all = zero_f
                    for r2 in range(c // 2):
                        v0 = buf[2 * r2, pl.ds(o, _LANES)]
                        v1 = buf[2 * r2 + 1, pl.ds(o, _LANES)]
                        v01 = v0 + v1
                        s_all = s_all + v01
                        s_head = s_head + jnp.where(
                            jnp.full((_LANES,), 2 * r2 + 1,
                                     jnp.int32) < t_v, v01, 0.0)
                    strag = plsc.load_gather(
                        buf, [strag_row, o + lane_iota])
                    s_head = s_head + jnp.where(odd, strag, 0.0)
                    plsc.addupdate_scatter(acc, [base1_v + o], s_head)
                    plsc.addupdate_scatter(acc, [base2_v + o],
                                           s_all - s_head)
                    return 0
                lax.fori_loop(0, d // _LANES, cc_body, 0)

            def tail():
                t2 = jnp.clip(r1 - s0, 0, c)
                t2_v = jnp.full((_LANES,), t2, jnp.int32)
                t1_v = jnp.minimum(t_v, t2_v)
                strag1_row = jnp.maximum(t1_v - 1, 0)
                odd1 = (t1_v & 1) == 1
                strag2_row = jnp.maximum(t2_v - 1, 0)
                odd2 = (t2_v & 1) == 1

                def cc_body(cc, _):
                    o = cc * _LANES
                    s1 = zero_f
                    s2 = zero_f
                    for r2 in range(c // 2):
                        v0 = buf[2 * r2, pl.ds(o, _LANES)]
                        v1 = buf[2 * r2 + 1, pl.ds(o, _LANES)]
                        v01 = v0 + v1
                        p_v = jnp.full((_LANES,), 2 * r2 + 1, jnp.int32)
                        s1 = s1 + jnp.where(p_v < t1_v, v01, 0.0)
                        s2 = s2 + jnp.where(p_v < t2_v, v01, 0.0)
                    g1 = plsc.load_gather(buf, [strag1_row, o + lane_iota])
                    s1 = s1 + jnp.where(odd1, g1, 0.0)
                    g2 = plsc.load_gather(buf, [strag2_row, o + lane_iota])
                    s2 = s2 + jnp.where(odd2, g2, 0.0)
                    plsc.addupdate_scatter(acc, [base1_v + o], s1)
                    plsc.addupdate_scatter(acc, [base2_v + o], s2 - s1)
                    return 0
                lax.fori_loop(0, d // _LANES, cc_body, 0)

            lax.cond(k < nch - 1, interior, tail)

        @pl.when(nch > 0)
        def _():
            start(idx0, buf0, sem0, jnp.int32(0))

        def pair(k2, _):
            k0 = k2 * 2
            k1 = k0 + 1

            @pl.when(k1 < nch)
            def _():
                start(idx1, buf1, sem1, k1)
            drain(buf0, sem0)
            process(buf0, k0)

            @pl.when(k0 + 2 < nch)
            def _():
                start(idx0, buf0, sem0, k0 + 2)

            @pl.when(k1 < nch)
            def _():
                drain(buf1, sem1)
                process(buf1, k1)
            return 0
        lax.fori_loop(0, (nch + 1) // 2, pair, 0)

        pltpu.sync_copy(acc.at[pl.ds(0, nseg * d)], out_hbm.at[wid])

    return sc_kernel(sent_embs, cu_bcast)


_TC_ROWS = 2048
_SC_FRACTION = 0.25


def _tc_body(cu_beg_ref, cu_end_ref, x_ref, p_ref, be_ref, len_ref,
             w_ref, o_ref, *, rows_per_step):
    step = pl.program_id(0)
    nsteps = pl.num_programs(0)
    r = rows_per_step
    rows = step * r + jax.lax.broadcasted_iota(jnp.int32, (1, r), 1)
    onehot = ((rows >= cu_beg_ref[:, :]) &
              (rows < cu_end_ref[:, :])).astype(jnp.float32)
    partial = jnp.dot(onehot, x_ref[:, :],
                      preferred_element_type=jnp.float32)

    @pl.when(step == 0)
    def _():
        o_ref[:, :] = jnp.zeros_like(o_ref)

    o_ref[:, :] += partial

    @pl.when(step == nsteps - 1)
    def _():
        s = o_ref[:, :] + jnp.sum(p_ref[...], axis=0)
        pooled = (s + be_ref[...]) / len_ref[...]
        o_ref[:, :] = jnp.dot(pooled, w_ref[...],
                              preferred_element_type=jnp.float32)


def kernel(sent_embs, frag_lengths, beg_seq_param, end_seq_param, W_enc):
    total, d = sent_embs.shape
    b = frag_lengths.shape[0]

    cu16 = jnp.cumsum(frag_lengths).astype(jnp.int32)
    cu_bcast = jnp.broadcast_to(cu16[:, None], (b, _LANES))
    cu = jnp.concatenate([jnp.zeros((1,), jnp.int32), cu16])
    cu_beg = cu[:b].reshape(b, 1)
    cu_end = cu[1:].reshape(b, 1)

    nblk = max(min(int(round((1.0 - _SC_FRACTION) * total / _TC_ROWS)),
                   total // _TC_ROWS), 1)
    split = nblk * _TC_ROWS

    partials = _sc_segment_partials(sent_embs, cu_bcast, split)
    partials = partials.reshape(_NW, b, d)

    be = (beg_seq_param + end_seq_param).reshape(1, d)
    len2 = (frag_lengths + 2).astype(jnp.float32).reshape(b, 1)

    return pl.pallas_call(
        functools.partial(_tc_body, rows_per_step=_TC_ROWS),
        grid=(nblk,),
        in_specs=[
            pl.BlockSpec((b, 1), lambda i: (0, 0)),
            pl.BlockSpec((b, 1), lambda i: (0, 0)),
            pl.BlockSpec((_TC_ROWS, d), lambda i: (i, 0)),
            pl.BlockSpec((_NW, b, d), lambda i: (0, 0, 0)),
            pl.BlockSpec((1, d), lambda i: (0, 0)),
            pl.BlockSpec((b, 1), lambda i: (0, 0)),
            pl.BlockSpec((d, d), lambda i: (0, 0)),
        ],
        out_specs=pl.BlockSpec((b, d), lambda i: (0, 0)),
        out_shape=jax.ShapeDtypeStruct((b, d), jnp.float32),
    )(cu_beg, cu_end, sent_embs, partials, be, len2, W_enc)


# speedup vs baseline: 1.1437x; 1.1437x over previous
import functools

import jax
import jax.numpy as jnp
from jax import lax
from jax.experimental import pallas as pl
from jax.experimental.pallas import tpu as pltpu
from jax.experimental.pallas import tpu_sc as plsc

_NC = 2
_NS = 16
_NW = _NC * _NS
_LANES = 16
_CHUNK = 48


def _sc_segment_partials(sent_embs, cu_bcast, row_begin):
    total, d = sent_embs.shape
    nseg = cu_bcast.shape[0]
    c = _CHUNK
    rpw = -(-(total - row_begin) // _NW)
    mesh = plsc.VectorSubcoreMesh(core_axis_name="c", subcore_axis_name="s")

    @functools.partial(
        pl.kernel,
        out_type=jax.ShapeDtypeStruct((_NW, nseg * d), jnp.float32),
        mesh=mesh,
        compiler_params=pltpu.CompilerParams(needs_layout_passes=False),
        scratch_types=[
            pltpu.VMEM(((nseg + 1) * d,), jnp.float32),
            pltpu.VMEM((nseg, _LANES), jnp.int32),
            pltpu.VMEM((c, d), jnp.float32),
            pltpu.VMEM((c, d), jnp.float32),
            pltpu.VMEM((c,), jnp.int32),
            pltpu.VMEM((c,), jnp.int32),
            pltpu.SemaphoreType.DMA,
            pltpu.SemaphoreType.DMA,
        ],
    )
    def sc_kernel(x_hbm, cu_hbm, out_hbm, acc, cu_b, buf0, buf1,
                  idx0, idx1, sem0, sem1):
        wid = lax.axis_index("s") * _NC + lax.axis_index("c")
        r0 = row_begin + wid * rpw
        r1 = jnp.minimum(total, r0 + rpw)
        nrows = jnp.maximum(r1 - r0, 0)
        nch = (nrows + c - 1) // c

        pltpu.sync_copy(cu_hbm, cu_b)
        lane_iota = lax.iota(jnp.int32, _LANES)
        c_v = jnp.full((_LANES,), c, jnp.int32)
        zero_f = jnp.zeros((_LANES,), jnp.float32)

        def zero_body(i, _):
            for u in range(16):
                acc[pl.ds((i * 16 + u) * _LANES, _LANES)] = zero_f
            return 0
        lax.fori_loop(0, (nseg + 1) * d // (_LANES * 16), zero_body, 0)

        def start(idx_ref, buf, sem, k):
            s0 = r0 + k * c
            for j in range(c // _LANES):
                idx_ref[pl.ds(j * _LANES, _LANES)] = jnp.minimum(
                    s0 + j * _LANES + lane_iota, total - 1)
            pltpu.async_copy(x_hbm.at[idx_ref], buf, sem)

        def drain(buf, sem):
            pltpu.make_async_copy(x_hbm.at[pl.ds(0, c)], buf, sem).wait()

        def process(buf, k):
            s0 = r0 + k * c
            s0_v = jnp.full((_LANES,), s0, jnp.int32)
            seg1_v = jnp.zeros((_LANES,), jnp.int32)
            t_v = c_v
            for j in range(nseg):
                cuj = cu_b[j, :]
                gt = cuj > s0_v
                seg1_v = seg1_v + jnp.where(gt, 0, 1)
                t_v = jnp.minimum(t_v, jnp.where(gt, cuj - s0_v, c_v))
            base1_v = seg1_v * d + lane_iota
            base2_v = jnp.minimum(seg1_v + 1, nseg) * d + lane_iota

            def interior():
                strag_row = jnp.maximum(t_v - 1, 0)
                odd = (t_v & 1) == 1

                def cc_body(cc, _):
                    o = cc * _LANES
                    s_head = zero_f
                    s_all = zero_f
                    for r2 in range(c // 2):
                        v0 = buf[2 * r2, pl.ds(o, _LANES)]
                        v1 = buf[2 * r2 + 1, pl.ds(o, _LANES)]
                        v01 = v0 + v1
                        s_all = s_all + v01
                        s_head = s_head + jnp.where(
                            jnp.full((_LANES,), 2 * r2 + 1,
                                     jnp.int32) < t_v, v01, 0.0)
                    strag = plsc.load_gather(
                        buf, [strag_row, o + lane_iota])
                    s_head = s_head + jnp.where(odd, strag, 0.0)
                    plsc.addupdate_scatter(acc, [base1_v + o], s_head)
                    plsc.addupdate_scatter(acc, [base2_v + o],
                                           s_all - s_head)
                    return 0
                lax.fori_loop(0, d // _LANES, cc_body, 0)

            def tail():
                t2 = jnp.clip(r1 - s0, 0, c)
                t2_v = jnp.full((_LANES,), t2, jnp.int32)
                t1_v = jnp.minimum(t_v, t2_v)
                strag1_row = jnp.maximum(t1_v - 1, 0)
                odd1 = (t1_v & 1) == 1
                strag2_row = jnp.maximum(t2_v - 1, 0)
                odd2 = (t2_v & 1) == 1

                def cc_body(cc, _):
                    o = cc * _LANES
                    s1 = zero_f
                    s2 = zero_f
                    for r2 in range(c // 2):
                        v0 = buf[2 * r2, pl.ds(o, _LANES)]
                        v1 = buf[2 * r2 + 1, pl.ds(o, _LANES)]
                        v01 = v0 + v1
                        p_v = jnp.full((_LANES,), 2 * r2 + 1, jnp.int32)
                        s1 = s1 + jnp.where(p_v < t1_v, v01, 0.0)
                        s2 = s2 + jnp.where(p_v < t2_v, v01, 0.0)
                    g1 = plsc.load_gather(buf, [strag1_row, o + lane_iota])
                    s1 = s1 + jnp.where(odd1, g1, 0.0)
                    g2 = plsc.load_gather(buf, [strag2_row, o + lane_iota])
                    s2 = s2 + jnp.where(odd2, g2, 0.0)
                    plsc.addupdate_scatter(acc, [base1_v + o], s1)
                    plsc.addupdate_scatter(acc, [base2_v + o], s2 - s1)
                    return 0
                lax.fori_loop(0, d // _LANES, cc_body, 0)

            lax.cond(k < nch - 1, interior, tail)

        @pl.when(nch > 0)
        def _():
            start(idx0, buf0, sem0, jnp.int32(0))

        def pair(k2, _):
            k0 = k2 * 2
            k1 = k0 + 1

            @pl.when(k1 < nch)
            def _():
                start(idx1, buf1, sem1, k1)
            drain(buf0, sem0)
            process(buf0, k0)

            @pl.when(k0 + 2 < nch)
            def _():
                start(idx0, buf0, sem0, k0 + 2)

            @pl.when(k1 < nch)
            def _():
                drain(buf1, sem1)
                process(buf1, k1)
            return 0
        lax.fori_loop(0, (nch + 1) // 2, pair, 0)

        pltpu.sync_copy(acc.at[pl.ds(0, nseg * d)], out_hbm.at[wid])

    return sc_kernel(sent_embs, cu_bcast)


_TC_ROWS = 2048
_SC_FRACTION = 0.25


def _tc_partial_body(cu_beg_ref, cu_end_ref, x_ref, o_ref, *,
                     rows_per_step):
    step = pl.program_id(0)
    r = rows_per_step
    rows = step * r + jax.lax.broadcasted_iota(jnp.int32, (1, r), 1)
    onehot = ((rows >= cu_beg_ref[:, :]) &
              (rows < cu_end_ref[:, :])).astype(jnp.float32)
    partial = jnp.dot(onehot, x_ref[:, :],
                      preferred_element_type=jnp.float32)

    @pl.when(step == 0)
    def _():
        o_ref[:, :] = jnp.zeros_like(o_ref)

    o_ref[:, :] += partial


def _finalize_body(p_ref, tc_ref, be_ref, len_ref, w_ref, o_ref):
    s = jnp.sum(p_ref[...], axis=0) + tc_ref[...]
    pooled = (s + be_ref[...]) / len_ref[...]
    o_ref[...] = jnp.dot(pooled, w_ref[...],
                         preferred_element_type=jnp.float32)


def kernel(sent_embs, frag_lengths, beg_seq_param, end_seq_param, W_enc):
    total, d = sent_embs.shape
    b = frag_lengths.shape[0]

    cu16 = jnp.cumsum(frag_lengths).astype(jnp.int32)
    cu_bcast = jnp.broadcast_to(cu16[:, None], (b, _LANES))
    cu = jnp.concatenate([jnp.zeros((1,), jnp.int32), cu16])
    cu_beg = cu[:b].reshape(b, 1)
    cu_end = cu[1:].reshape(b, 1)

    nblk = max(min(int(round((1.0 - _SC_FRACTION) * total / _TC_ROWS)),
                   total // _TC_ROWS), 1)
    split = nblk * _TC_ROWS

    tc_partial = pl.pallas_call(
        functools.partial(_tc_partial_body, rows_per_step=_TC_ROWS),
        grid=(nblk,),
        in_specs=[
            pl.BlockSpec((b, 1), lambda i: (0, 0)),
            pl.BlockSpec((b, 1), lambda i: (0, 0)),
            pl.BlockSpec((_TC_ROWS, d), lambda i: (i, 0)),
        ],
        out_specs=pl.BlockSpec((b, d), lambda i: (0, 0)),
        out_shape=jax.ShapeDtypeStruct((b, d), jnp.float32),
    )(cu_beg, cu_end, sent_embs)

    partials = _sc_segment_partials(sent_embs, cu_bcast, split)
    partials = partials.reshape(_NW, b, d)

    be = (beg_seq_param + end_seq_param).reshape(1, d)
    len2 = (frag_lengths + 2).astype(jnp.float32).reshape(b, 1)

    return pl.pallas_call(
        _finalize_body,
        in_specs=[
            pl.BlockSpec((_NW, b, d), lambda: (0, 0, 0)),
            pl.BlockSpec((b, d), lambda: (0, 0)),
            pl.BlockSpec((1, d), lambda: (0, 0)),
            pl.BlockSpec((b, 1), lambda: (0, 0)),
            pl.BlockSpec((d, d), lambda: (0, 0)),
        ],
        out_specs=pl.BlockSpec((b, d), lambda: (0, 0)),
        out_shape=jax.ShapeDtypeStruct((b, d), jnp.float32),
    )(partials, tc_partial, be, len2, W_enc)


# speedup vs baseline: 1.1636x; 1.0174x over previous
import functools

import jax
import jax.numpy as jnp
from jax import lax
from jax.experimental import pallas as pl
from jax.experimental.pallas import tpu as pltpu
from jax.experimental.pallas import tpu_sc as plsc

_NC = 2
_NS = 16
_NW = _NC * _NS
_LANES = 16
_CHUNK = 48


def _sc_segment_partials(sent_embs, cu_bcast, row_begin):
    total, d = sent_embs.shape
    nseg = cu_bcast.shape[0]
    c = _CHUNK
    rpw = -(-(total - row_begin) // _NW)
    mesh = plsc.VectorSubcoreMesh(core_axis_name="c", subcore_axis_name="s")

    @functools.partial(
        pl.kernel,
        out_type=jax.ShapeDtypeStruct((_NW, nseg * d), jnp.float32),
        mesh=mesh,
        compiler_params=pltpu.CompilerParams(needs_layout_passes=False),
        scratch_types=[
            pltpu.VMEM(((nseg + 1) * d,), jnp.float32),
            pltpu.VMEM((nseg, _LANES), jnp.int32),
            pltpu.VMEM((c, d), jnp.float32),
            pltpu.VMEM((c, d), jnp.float32),
            pltpu.VMEM((c,), jnp.int32),
            pltpu.VMEM((c,), jnp.int32),
            pltpu.SemaphoreType.DMA,
            pltpu.SemaphoreType.DMA,
        ],
    )
    def sc_kernel(x_hbm, cu_hbm, out_hbm, acc, cu_b, buf0, buf1,
                  idx0, idx1, sem0, sem1):
        wid = lax.axis_index("s") * _NC + lax.axis_index("c")
        r0 = row_begin + wid * rpw
        r1 = jnp.minimum(total, r0 + rpw)
        nrows = jnp.maximum(r1 - r0, 0)
        nch = (nrows + c - 1) // c

        pltpu.sync_copy(cu_hbm, cu_b)
        lane_iota = lax.iota(jnp.int32, _LANES)
        c_v = jnp.full((_LANES,), c, jnp.int32)
        zero_f = jnp.zeros((_LANES,), jnp.float32)

        def zero_body(i, _):
            for u in range(16):
                acc[pl.ds((i * 16 + u) * _LANES, _LANES)] = zero_f
            return 0
        lax.fori_loop(0, (nseg + 1) * d // (_LANES * 16), zero_body, 0)

        def start(idx_ref, buf, sem, k):
            s0 = r0 + k * c
            for j in range(c // _LANES):
                idx_ref[pl.ds(j * _LANES, _LANES)] = jnp.minimum(
                    s0 + j * _LANES + lane_iota, total - 1)
            pltpu.async_copy(x_hbm.at[idx_ref], buf, sem)

        def drain(buf, sem):
            pltpu.make_async_copy(x_hbm.at[pl.ds(0, c)], buf, sem).wait()

        def process(buf, k):
            s0 = r0 + k * c
            s0_v = jnp.full((_LANES,), s0, jnp.int32)
            seg1_v = jnp.zeros((_LANES,), jnp.int32)
            t_v = c_v
            for j in range(nseg):
                cuj = cu_b[j, :]
                gt = cuj > s0_v
                seg1_v = seg1_v + jnp.where(gt, 0, 1)
                t_v = jnp.minimum(t_v, jnp.where(gt, cuj - s0_v, c_v))
            base1_v = seg1_v * d + lane_iota
            base2_v = jnp.minimum(seg1_v + 1, nseg) * d + lane_iota

            def interior():
                strag_row = jnp.maximum(t_v - 1, 0)
                odd = (t_v & 1) == 1

                def cc_body(cc, _):
                    o = cc * _LANES
                    s_head = zero_f
                    s_all = zero_f
                    for r2 in range(c // 2):
                        v0 = buf[2 * r2, pl.ds(o, _LANES)]
                        v1 = buf[2 * r2 + 1, pl.ds(o, _LANES)]
                        v01 = v0 + v1
                        s_all = s_all + v01
                        s_head = s_head + jnp.where(
                            jnp.full((_LANES,), 2 * r2 + 1,
                                     jnp.int32) < t_v, v01, 0.0)
                    strag = plsc.load_gather(
                        buf, [strag_row, o + lane_iota])
                    s_head = s_head + jnp.where(odd, strag, 0.0)
                    plsc.addupdate_scatter(acc, [base1_v + o], s_head)
                    plsc.addupdate_scatter(acc, [base2_v + o],
                                           s_all - s_head)
                    return 0
                lax.fori_loop(0, d // _LANES, cc_body, 0)

            def tail():
                t2 = jnp.clip(r1 - s0, 0, c)
                t2_v = jnp.full((_LANES,), t2, jnp.int32)
                t1_v = jnp.minimum(t_v, t2_v)
                strag1_row = jnp.maximum(t1_v - 1, 0)
                odd1 = (t1_v & 1) == 1
                strag2_row = jnp.maximum(t2_v - 1, 0)
                odd2 = (t2_v & 1) == 1

                def cc_body(cc, _):
                    o = cc * _LANES
                    s1 = zero_f
                    s2 = zero_f
                    for r2 in range(c // 2):
                        v0 = buf[2 * r2, pl.ds(o, _LANES)]
                        v1 = buf[2 * r2 + 1, pl.ds(o, _LANES)]
                        v01 = v0 + v1
                        p_v = jnp.full((_LANES,), 2 * r2 + 1, jnp.int32)
                        s1 = s1 + jnp.where(p_v < t1_v, v01, 0.0)
                        s2 = s2 + jnp.where(p_v < t2_v, v01, 0.0)
                    g1 = plsc.load_gather(buf, [strag1_row, o + lane_iota])
                    s1 = s1 + jnp.where(odd1, g1, 0.0)
                    g2 = plsc.load_gather(buf, [strag2_row, o + lane_iota])
                    s2 = s2 + jnp.where(odd2, g2, 0.0)
                    plsc.addupdate_scatter(acc, [base1_v + o], s1)
                    plsc.addupdate_scatter(acc, [base2_v + o], s2 - s1)
                    return 0
                lax.fori_loop(0, d // _LANES, cc_body, 0)

            lax.cond(k < nch - 1, interior, tail)

        @pl.when(nch > 0)
        def _():
            start(idx0, buf0, sem0, jnp.int32(0))

        def pair(k2, _):
            k0 = k2 * 2
            k1 = k0 + 1

            @pl.when(k1 < nch)
            def _():
                start(idx1, buf1, sem1, k1)
            drain(buf0, sem0)
            process(buf0, k0)

            @pl.when(k0 + 2 < nch)
            def _():
                start(idx0, buf0, sem0, k0 + 2)

            @pl.when(k1 < nch)
            def _():
                drain(buf1, sem1)
                process(buf1, k1)
            return 0
        lax.fori_loop(0, (nch + 1) // 2, pair, 0)

        pltpu.sync_copy(acc.at[pl.ds(0, nseg * d)], out_hbm.at[wid])

    return sc_kernel(sent_embs, cu_bcast)


_TC_ROWS = 2048
_SC_FRACTION = 0.20


def _tc_partial_body(cu_beg_ref, cu_end_ref, x_ref, o_ref, *,
                     rows_per_step):
    step = pl.program_id(0)
    r = rows_per_step
    rows = step * r + jax.lax.broadcasted_iota(jnp.int32, (1, r), 1)
    onehot = ((rows >= cu_beg_ref[:, :]) &
              (rows < cu_end_ref[:, :])).astype(jnp.float32)
    partial = jnp.dot(onehot, x_ref[:, :],
                      preferred_element_type=jnp.float32)

    @pl.when(step == 0)
    def _():
        o_ref[:, :] = jnp.zeros_like(o_ref)

    o_ref[:, :] += partial


def _finalize_body(p_ref, tc_ref, be_ref, len_ref, w_ref, o_ref):
    s = jnp.sum(p_ref[...], axis=0) + tc_ref[...]
    pooled = (s + be_ref[...]) / len_ref[...]
    o_ref[...] = jnp.dot(pooled, w_ref[...],
                         preferred_element_type=jnp.float32)


def kernel(sent_embs, frag_lengths, beg_seq_param, end_seq_param, W_enc):
    total, d = sent_embs.shape
    b = frag_lengths.shape[0]

    cu16 = jnp.cumsum(frag_lengths).astype(jnp.int32)
    cu_bcast = jnp.broadcast_to(cu16[:, None], (b, _LANES))
    cu = jnp.concatenate([jnp.zeros((1,), jnp.int32), cu16])
    cu_beg = cu[:b].reshape(b, 1)
    cu_end = cu[1:].reshape(b, 1)

    nblk = max(min(int(round((1.0 - _SC_FRACTION) * total / _TC_ROWS)),
                   total // _TC_ROWS), 1)
    split = nblk * _TC_ROWS

    tc_partial = pl.pallas_call(
        functools.partial(_tc_partial_body, rows_per_step=_TC_ROWS),
        grid=(nblk,),
        in_specs=[
            pl.BlockSpec((b, 1), lambda i: (0, 0)),
            pl.BlockSpec((b, 1), lambda i: (0, 0)),
            pl.BlockSpec((_TC_ROWS, d), lambda i: (i, 0)),
        ],
        out_specs=pl.BlockSpec((b, d), lambda i: (0, 0)),
        out_shape=jax.ShapeDtypeStruct((b, d), jnp.float32),
    )(cu_beg, cu_end, sent_embs)

    partials = _sc_segment_partials(sent_embs, cu_bcast, split)
    partials = partials.reshape(_NW, b, d)

    be = (beg_seq_param + end_seq_param).reshape(1, d)
    len2 = (frag_lengths + 2).astype(jnp.float32).reshape(b, 1)

    return pl.pallas_call(
        _finalize_body,
        in_specs=[
            pl.BlockSpec((_NW, b, d), lambda: (0, 0, 0)),
            pl.BlockSpec((b, d), lambda: (0, 0)),
            pl.BlockSpec((1, d), lambda: (0, 0)),
            pl.BlockSpec((b, 1), lambda: (0, 0)),
            pl.BlockSpec((d, d), lambda: (0, 0)),
        ],
        out_specs=pl.BlockSpec((b, d), lambda: (0, 0)),
        out_shape=jax.ShapeDtypeStruct((b, d), jnp.float32),
    )(partials, tc_partial, be, len2, W_enc)
